# baseline (speedup 1.0000x reference)
import functools

import jax
import jax.numpy as jnp
from jax import lax
from jax.experimental import pallas as pl
from jax.experimental.pallas import tpu as pltpu
from jax.experimental.pallas import tpu_sc as plsc

N = 10000
E = 320000
D = 128
H = 16
NC = 2
NS = 16
NW = NC * NS

CH = 128
KB = 8
KC0 = 160
KC1 = 0
KCM = max(KC0, KC1)
NCH = NS * (KC0 + KC1)
EP = NCH * CH
NR = 10240
RPT = NR // NS

_mesh = plsc.VectorSubcoreMesh(core_axis_name="c", subcore_axis_name="s")


@functools.partial(
    pl.kernel,
    out_type=jax.ShapeDtypeStruct((NC, NR), jnp.float32),
    mesh=_mesh,
    scratch_types=[
        pltpu.VMEM((KCM, CH), jnp.int32),
        pltpu.VMEM((CH,), jnp.float32),
        pltpu.VMEM_SHARED((NR,), jnp.float32),
        pltpu.SemaphoreType.DMA,
    ],
)
def _sc_degree(dstp_hbm, zeros1_hbm, out_hbm, dst_v, ones_v, deg_sh, sem):
    c = lax.axis_index("c")
    s = lax.axis_index("s")
    pltpu.sync_copy(zeros1_hbm.at[pl.ds(s * RPT, RPT)],
                    deg_sh.at[pl.ds(s * RPT, RPT)])
    for i in range(CH // 16):
        ones_v[pl.ds(i * 16, 16)] = jnp.ones((16,), jnp.float32)

    def run(kc, cbase):
        if kc == 0:
            return
        pltpu.sync_copy(dstp_hbm.at[pl.ds(cbase, kc)],
                        dst_v.at[pl.ds(0, kc)])
        descs = [pltpu.async_copy(
            ones_v, deg_sh.at[dst_v.at[j]], sem, add=True)
            for j in range(kc)]
        for d in descs:
            d.wait()

    plsc.subcore_barrier()
    lax.cond(c == 0,
             lambda: run(KC0, s * KC0),
             lambda: run(KC1, NS * KC0 + s * KC1))
    plsc.subcore_barrier()
    pltpu.sync_copy(deg_sh.at[pl.ds(s * RPT, RPT)],
                    out_hbm.at[c].at[pl.ds(s * RPT, RPT)])


@functools.partial(
    pl.kernel,
    out_type=jax.ShapeDtypeStruct((NC, NR, H), jnp.float32),
    mesh=_mesh,
    compiler_params=pltpu.CompilerParams(use_tc_tiling_on_sc=False),
    scratch_types=[
        pltpu.VMEM((KCM, CH), jnp.int32),
        pltpu.VMEM((KCM, CH), jnp.int32),
        pltpu.VMEM((2, KB, CH, H), jnp.float32),
        pltpu.VMEM_SHARED((NR, H), jnp.float32),
        pltpu.SemaphoreType.DMA,
        pltpu.SemaphoreType.DMA,
    ],
)
def _sc_aggregate(g_hbm, srcp_hbm, dstp_hbm, zeros2_hbm, out_hbm,
                  src_v, dst_v, rows_v, acc_sh, gsem, ssem):
    c = lax.axis_index("c")
    s = lax.axis_index("s")
    pltpu.sync_copy(zeros2_hbm.at[pl.ds(s * RPT, RPT)],
                    acc_sh.at[pl.ds(s * RPT, RPT)])

    def gather_group(gi, bank, kb):
        return [pltpu.async_copy(
            g_hbm.at[src_v.at[gi * KB + b]], rows_v.at[bank].at[b], gsem)
            for b in range(kb)]

    def scatter_group(gi, bank, kb):
        return [pltpu.async_copy(
            rows_v.at[bank].at[b], acc_sh.at[dst_v.at[gi * KB + b]],
            ssem, add=True)
            for b in range(kb)]

    def run(kc, cbase):
        if kc == 0:
            return
        pltpu.sync_copy(srcp_hbm.at[pl.ds(cbase, kc)],
                        src_v.at[pl.ds(0, kc)])
        pltpu.sync_copy(dstp_hbm.at[pl.ds(cbase, kc)],
                        dst_v.at[pl.ds(0, kc)])
        ng = kc // KB
        gd = gather_group(0, 0, KB)
        pending = None
        for gi in range(ng):
            bank = gi % 2
            for d in gd:
                d.wait()
            if gi + 1 < ng:
                if pending is not None:
                    for d in pending:
                        d.wait()
                gd = gather_group(gi + 1, 1 - bank, KB)
                pending = scatter_group(gi, bank, KB)
            else:
                last = scatter_group(gi, bank, KB)
                for d in pending:
                    d.wait()
                for d in last:
                    d.wait()

    plsc.subcore_barrier()
    lax.cond(c == 0,
             lambda: run(KC0, s * KC0),
             lambda: run(KC1, NS * KC0 + s * KC1))
    plsc.subcore_barrier()
    pltpu.sync_copy(acc_sh.at[pl.ds(s * RPT, RPT)],
                    out_hbm.at[c].at[pl.ds(s * RPT, RPT)])


def _dense0_body(x_ref, w_ref, deg_ref, g_ref, dinv_ref):
    d = deg_ref[0] + deg_ref[1] + 1.0
    dinv = lax.rsqrt(d)[:N]
    h = jnp.dot(x_ref[...], w_ref[...], preferred_element_type=jnp.float32)
    g_ref[...] = dinv * h
    dinv_ref[...] = dinv


def _layer_body(acc_ref, g_ref, dinv_ref, b_ref, w_ref, out_ref):
    dinv = dinv_ref[...]
    f = jnp.maximum(
        dinv * (acc_ref[0, :N] + acc_ref[1, :N] + g_ref[...]) + b_ref[...],
        0.0)
    out_ref[...] = dinv * jnp.dot(f, w_ref[...],
                                  preferred_element_type=jnp.float32)


def _layer4_body(acc_ref, g_ref, dinv_ref, b_ref, out_ref):
    dinv = dinv_ref[...]
    out_ref[...] = dinv * jnp.maximum(
        dinv * (acc_ref[0, :N] + acc_ref[1, :N] + g_ref[...]) + b_ref[...],
        0.0)


def _final_body(acc_ref, g_ref, dinv_ref, w_ref, b_ref, out_ref):
    af = dinv_ref[...] * (acc_ref[0, :N] + acc_ref[1, :N] + g_ref[...])
    z = jnp.dot(af, w_ref[...],
                preferred_element_type=jnp.float32) + b_ref[...]
    z = z.reshape(100, 100)
    m = jnp.max(z, axis=1, keepdims=True)
    e = jnp.exp(z - m)
    out_ref[...] = e / jnp.sum(e, axis=1, keepdims=True)


_f32 = jnp.float32

_dense0 = pl.pallas_call(
    _dense0_body,
    out_shape=(jax.ShapeDtypeStruct((N, H), _f32),
               jax.ShapeDtypeStruct((N, 1), _f32)))

_layer = pl.pallas_call(
    _layer_body,
    out_shape=jax.ShapeDtypeStruct((N, H), _f32))

_layer4 = pl.pallas_call(
    _layer4_body,
    out_shape=jax.ShapeDtypeStruct((N, H), _f32))

_final = pl.pallas_call(
    _final_body,
    out_shape=jax.ShapeDtypeStruct((100, 100), _f32))


def kernel(x, edge_index, W0, b0, W1, b1, W2, b2, W3, b3, W4, b4):
    src, dst = edge_index[0], edge_index[1]
    pad = EP - E
    srcp = jnp.concatenate(
        [src, jnp.zeros((pad,), jnp.int32)]).reshape(NCH, CH)
    pad_dst = N + (jnp.arange(pad, dtype=jnp.int32) % 128)
    dstp = jnp.concatenate([dst, pad_dst]).reshape(NCH, CH)
    zeros1 = jnp.zeros((NR,), _f32)
    zeros2 = jnp.zeros((NR, H), _f32)

    degp = _sc_degree(dstp, zeros1)
    g, dinv = _dense0(x, W0, degp.reshape(NC, NR, 1))

    for b, W in ((b0, W1), (b1, W2), (b2, W3)):
        acc = _sc_aggregate(g, srcp, dstp, zeros2)
        g = _layer(acc, g, dinv, b.reshape(1, H), W)

    acc = _sc_aggregate(g, srcp, dstp, zeros2)
    g = _layer4(acc, g, dinv, b3.reshape(1, H))

    acc = _sc_aggregate(g, srcp, dstp, zeros2)
    out = _final(acc, g, dinv, W4, b4.reshape(1, 1))
    return out.reshape(1, 100, 100)


# speedup vs baseline: 1.2030x; 1.2030x over previous
import functools

import jax
import jax.numpy as jnp
from jax import lax
from jax.experimental import pallas as pl
from jax.experimental.pallas import tpu as pltpu
from jax.experimental.pallas import tpu_sc as plsc

N = 10000
E = 320000
D = 128
H = 16
NC = 2
NS = 16
NW = NC * NS

CH = 128
KB = 8
KC0 = 96
KC1 = 64
KCM = max(KC0, KC1)
NCH = NS * (KC0 + KC1)
EP = NCH * CH
NR = 10240
RPT = NR // NS

_mesh = plsc.VectorSubcoreMesh(core_axis_name="c", subcore_axis_name="s")


@functools.partial(
    pl.kernel,
    out_type=jax.ShapeDtypeStruct((NC, NR), jnp.float32),
    mesh=_mesh,
    scratch_types=[
        pltpu.VMEM((KCM, CH), jnp.int32),
        pltpu.VMEM((CH,), jnp.float32),
        pltpu.VMEM_SHARED((NR,), jnp.float32),
        pltpu.SemaphoreType.DMA,
    ],
)
def _sc_degree(dstp_hbm, zeros1_hbm, out_hbm, dst_v, ones_v, deg_sh, sem):
    c = lax.axis_index("c")
    s = lax.axis_index("s")
    pltpu.sync_copy(zeros1_hbm.at[pl.ds(s * RPT, RPT)],
                    deg_sh.at[pl.ds(s * RPT, RPT)])
    for i in range(CH // 16):
        ones_v[pl.ds(i * 16, 16)] = jnp.ones((16,), jnp.float32)

    def run(kc, cbase):
        if kc == 0:
            return
        pltpu.sync_copy(dstp_hbm.at[pl.ds(cbase, kc)],
                        dst_v.at[pl.ds(0, kc)])
        descs = [pltpu.async_copy(
            ones_v, deg_sh.at[dst_v.at[j]], sem, add=True)
            for j in range(kc)]
        for d in descs:
            d.wait()

    plsc.subcore_barrier()
    lax.cond(c == 0,
             lambda: run(KC0, s * KC0),
             lambda: run(KC1, NS * KC0 + s * KC1))
    plsc.subcore_barrier()
    pltpu.sync_copy(deg_sh.at[pl.ds(s * RPT, RPT)],
                    out_hbm.at[c].at[pl.ds(s * RPT, RPT)])


@functools.partial(
    pl.kernel,
    out_type=jax.ShapeDtypeStruct((NC, NR, H), jnp.float32),
    mesh=_mesh,
    compiler_params=pltpu.CompilerParams(use_tc_tiling_on_sc=False),
    scratch_types=[
        pltpu.VMEM((KCM, CH), jnp.int32),
        pltpu.VMEM((KCM, CH), jnp.int32),
        pltpu.VMEM((2, KB, CH, H), jnp.float32),
        pltpu.VMEM_SHARED((NR, H), jnp.float32),
        pltpu.SemaphoreType.DMA,
        pltpu.SemaphoreType.DMA,
    ],
)
def _sc_aggregate(g_hbm, srcp_hbm, dstp_hbm, zeros2_hbm, out_hbm,
                  src_v, dst_v, rows_v, acc_sh, gsem, ssem):
    c = lax.axis_index("c")
    s = lax.axis_index("s")
    pltpu.sync_copy(zeros2_hbm.at[pl.ds(s * RPT, RPT)],
                    acc_sh.at[pl.ds(s * RPT, RPT)])

    def gather_group(gi, bank, kb):
        return [pltpu.async_copy(
            g_hbm.at[src_v.at[gi * KB + b]], rows_v.at[bank].at[b], gsem)
            for b in range(kb)]

    def scatter_group(gi, bank, kb):
        return [pltpu.async_copy(
            rows_v.at[bank].at[b], acc_sh.at[dst_v.at[gi * KB + b]],
            ssem, add=True)
            for b in range(kb)]

    def run(kc, cbase):
        if kc == 0:
            return
        pltpu.sync_copy(srcp_hbm.at[pl.ds(cbase, kc)],
                        src_v.at[pl.ds(0, kc)])
        pltpu.sync_copy(dstp_hbm.at[pl.ds(cbase, kc)],
                        dst_v.at[pl.ds(0, kc)])
        ng = kc // KB
        gd = gather_group(0, 0, KB)
        pending = None
        for gi in range(ng):
            bank = gi % 2
            for d in gd:
                d.wait()
            if gi + 1 < ng:
                if pending is not None:
                    for d in pending:
                        d.wait()
                gd = gather_group(gi + 1, 1 - bank, KB)
                pending = scatter_group(gi, bank, KB)
            else:
                last = scatter_group(gi, bank, KB)
                for d in pending:
                    d.wait()
                for d in last:
                    d.wait()

    plsc.subcore_barrier()
    lax.cond(c == 0,
             lambda: run(KC0, s * KC0),
             lambda: run(KC1, NS * KC0 + s * KC1))
    plsc.subcore_barrier()
    pltpu.sync_copy(acc_sh.at[pl.ds(s * RPT, RPT)],
                    out_hbm.at[c].at[pl.ds(s * RPT, RPT)])


def _dense0_body(x_ref, w_ref, deg_ref, g_ref, dinv_ref):
    d = deg_ref[0] + deg_ref[1] + 1.0
    dinv = lax.rsqrt(d)[:N]
    h = jnp.dot(x_ref[...], w_ref[...], preferred_element_type=jnp.float32)
    g_ref[...] = dinv * h
    dinv_ref[...] = dinv


def _layer_body(acc_ref, g_ref, dinv_ref, b_ref, w_ref, out_ref):
    dinv = dinv_ref[...]
    f = jnp.maximum(
        dinv * (acc_ref[0, :N] + acc_ref[1, :N] + g_ref[...]) + b_ref[...],
        0.0)
    out_ref[...] = dinv * jnp.dot(f, w_ref[...],
                                  preferred_element_type=jnp.float32)


def _layer4_body(acc_ref, g_ref, dinv_ref, b_ref, out_ref):
    dinv = dinv_ref[...]
    out_ref[...] = dinv * jnp.maximum(
        dinv * (acc_ref[0, :N] + acc_ref[1, :N] + g_ref[...]) + b_ref[...],
        0.0)


def _final_body(acc_ref, g_ref, dinv_ref, w_ref, b_ref, out_ref):
    af = dinv_ref[...] * (acc_ref[0, :N] + acc_ref[1, :N] + g_ref[...])
    z = jnp.dot(af, w_ref[...],
                preferred_element_type=jnp.float32) + b_ref[...]
    z = z.reshape(100, 100)
    m = jnp.max(z, axis=1, keepdims=True)
    e = jnp.exp(z - m)
    out_ref[...] = e / jnp.sum(e, axis=1, keepdims=True)


_f32 = jnp.float32

_dense0 = pl.pallas_call(
    _dense0_body,
    out_shape=(jax.ShapeDtypeStruct((N, H), _f32),
               jax.ShapeDtypeStruct((N, 1), _f32)))

_layer = pl.pallas_call(
    _layer_body,
    out_shape=jax.ShapeDtypeStruct((N, H), _f32))

_layer4 = pl.pallas_call(
    _layer4_body,
    out_shape=jax.ShapeDtypeStruct((N, H), _f32))

_final = pl.pallas_call(
    _final_body,
    out_shape=jax.ShapeDtypeStruct((100, 100), _f32))


def kernel(x, edge_index, W0, b0, W1, b1, W2, b2, W3, b3, W4, b4):
    src, dst = edge_index[0], edge_index[1]
    pad = EP - E
    srcp = jnp.concatenate(
        [src, jnp.zeros((pad,), jnp.int32)]).reshape(NCH, CH)
    pad_dst = N + (jnp.arange(pad, dtype=jnp.int32) % 128)
    dstp = jnp.concatenate([dst, pad_dst]).reshape(NCH, CH)
    zeros1 = jnp.zeros((NR,), _f32)
    zeros2 = jnp.zeros((NR, H), _f32)

    degp = _sc_degree(dstp, zeros1)
    g, dinv = _dense0(x, W0, degp.reshape(NC, NR, 1))

    for b, W in ((b0, W1), (b1, W2), (b2, W3)):
        acc = _sc_aggregate(g, srcp, dstp, zeros2)
        g = _layer(acc, g, dinv, b.reshape(1, H), W)

    acc = _sc_aggregate(g, srcp, dstp, zeros2)
    g = _layer4(acc, g, dinv, b3.reshape(1, H))

    acc = _sc_aggregate(g, srcp, dstp, zeros2)
    out = _final(acc, g, dinv, W4, b4.reshape(1, 1))
    return out.reshape(1, 100, 100)


# speedup vs baseline: 1.2529x; 1.0415x over previous
import functools

import jax
import jax.numpy as jnp
from jax import lax
from jax.experimental import pallas as pl
from jax.experimental.pallas import tpu as pltpu
from jax.experimental.pallas import tpu_sc as plsc

N = 10000
E = 320000
D = 128
H = 16
NC = 2
NS = 16
NW = NC * NS

CH = 128
KB = 8
KC0 = 128
KC1 = 32
KCM = max(KC0, KC1)
NCH = NS * (KC0 + KC1)
EP = NCH * CH
NR = 10240
RPT = NR // NS

_mesh = plsc.VectorSubcoreMesh(core_axis_name="c", subcore_axis_name="s")


@functools.partial(
    pl.kernel,
    out_type=jax.ShapeDtypeStruct((NC, NR), jnp.float32),
    mesh=_mesh,
    scratch_types=[
        pltpu.VMEM((KCM, CH), jnp.int32),
        pltpu.VMEM((CH,), jnp.float32),
        pltpu.VMEM_SHARED((NR,), jnp.float32),
        pltpu.SemaphoreType.DMA,
    ],
)
def _sc_degree(dstp_hbm, zeros1_hbm, out_hbm, dst_v, ones_v, deg_sh, sem):
    c = lax.axis_index("c")
    s = lax.axis_index("s")
    pltpu.sync_copy(zeros1_hbm.at[pl.ds(s * RPT, RPT)],
                    deg_sh.at[pl.ds(s * RPT, RPT)])
    for i in range(CH // 16):
        ones_v[pl.ds(i * 16, 16)] = jnp.ones((16,), jnp.float32)

    def run(kc, cbase):
        if kc == 0:
            return
        pltpu.sync_copy(dstp_hbm.at[pl.ds(cbase, kc)],
                        dst_v.at[pl.ds(0, kc)])
        descs = [pltpu.async_copy(
            ones_v, deg_sh.at[dst_v.at[j]], sem, add=True)
            for j in range(kc)]
        for d in descs:
            d.wait()

    plsc.subcore_barrier()
    lax.cond(c == 0,
             lambda: run(KC0, s * KC0),
             lambda: run(KC1, NS * KC0 + s * KC1))
    plsc.subcore_barrier()
    pltpu.sync_copy(deg_sh.at[pl.ds(s * RPT, RPT)],
                    out_hbm.at[c].at[pl.ds(s * RPT, RPT)])


@functools.partial(
    pl.kernel,
    out_type=jax.ShapeDtypeStruct((NC, NR, H), jnp.float32),
    mesh=_mesh,
    compiler_params=pltpu.CompilerParams(use_tc_tiling_on_sc=False),
    scratch_types=[
        pltpu.VMEM((KCM, CH), jnp.int32),
        pltpu.VMEM((KCM, CH), jnp.int32),
        pltpu.VMEM((2, KB, CH, H), jnp.float32),
        pltpu.VMEM_SHARED((NR, H), jnp.float32),
        pltpu.SemaphoreType.DMA,
        pltpu.SemaphoreType.DMA,
    ],
)
def _sc_aggregate(g_hbm, srcp_hbm, dstp_hbm, zeros2_hbm, out_hbm,
                  src_v, dst_v, rows_v, acc_sh, gsem, ssem):
    c = lax.axis_index("c")
    s = lax.axis_index("s")
    pltpu.sync_copy(zeros2_hbm.at[pl.ds(s * RPT, RPT)],
                    acc_sh.at[pl.ds(s * RPT, RPT)])

    def gather_group(gi, bank, kb):
        return [pltpu.async_copy(
            g_hbm.at[src_v.at[gi * KB + b]], rows_v.at[bank].at[b], gsem)
            for b in range(kb)]

    def scatter_group(gi, bank, kb):
        return [pltpu.async_copy(
            rows_v.at[bank].at[b], acc_sh.at[dst_v.at[gi * KB + b]],
            ssem, add=True)
            for b in range(kb)]

    def run(kc, cbase):
        if kc == 0:
            return
        pltpu.sync_copy(srcp_hbm.at[pl.ds(cbase, kc)],
                        src_v.at[pl.ds(0, kc)])
        pltpu.sync_copy(dstp_hbm.at[pl.ds(cbase, kc)],
                        dst_v.at[pl.ds(0, kc)])
        ng = kc // KB
        gd = gather_group(0, 0, KB)
        pending = None
        for gi in range(ng):
            bank = gi % 2
            for d in gd:
                d.wait()
            if gi + 1 < ng:
                if pending is not None:
                    for d in pending:
                        d.wait()
                gd = gather_group(gi + 1, 1 - bank, KB)
                pending = scatter_group(gi, bank, KB)
            else:
                last = scatter_group(gi, bank, KB)
                for d in pending:
                    d.wait()
                for d in last:
                    d.wait()

    plsc.subcore_barrier()
    lax.cond(c == 0,
             lambda: run(KC0, s * KC0),
             lambda: run(KC1, NS * KC0 + s * KC1))
    plsc.subcore_barrier()
    pltpu.sync_copy(acc_sh.at[pl.ds(s * RPT, RPT)],
                    out_hbm.at[c].at[pl.ds(s * RPT, RPT)])


def _dense0_body(x_ref, w_ref, deg_ref, g_ref, dinv_ref):
    d = deg_ref[0] + deg_ref[1] + 1.0
    dinv = lax.rsqrt(d)[:N]
    h = jnp.dot(x_ref[...], w_ref[...], preferred_element_type=jnp.float32)
    g_ref[...] = dinv * h
    dinv_ref[...] = dinv


def _layer_body(acc_ref, g_ref, dinv_ref, b_ref, w_ref, out_ref):
    dinv = dinv_ref[...]
    f = jnp.maximum(
        dinv * (acc_ref[0, :N] + acc_ref[1, :N] + g_ref[...]) + b_ref[...],
        0.0)
    out_ref[...] = dinv * jnp.dot(f, w_ref[...],
                                  preferred_element_type=jnp.float32)


def _layer4_body(acc_ref, g_ref, dinv_ref, b_ref, out_ref):
    dinv = dinv_ref[...]
    out_ref[...] = dinv * jnp.maximum(
        dinv * (acc_ref[0, :N] + acc_ref[1, :N] + g_ref[...]) + b_ref[...],
        0.0)


def _final_body(acc_ref, g_ref, dinv_ref, w_ref, b_ref, out_ref):
    af = dinv_ref[...] * (acc_ref[0, :N] + acc_ref[1, :N] + g_ref[...])
    z = jnp.dot(af, w_ref[...],
                preferred_element_type=jnp.float32) + b_ref[...]
    z = z.reshape(100, 100)
    m = jnp.max(z, axis=1, keepdims=True)
    e = jnp.exp(z - m)
    out_ref[...] = e / jnp.sum(e, axis=1, keepdims=True)


_f32 = jnp.float32

_dense0 = pl.pallas_call(
    _dense0_body,
    out_shape=(jax.ShapeDtypeStruct((N, H), _f32),
               jax.ShapeDtypeStruct((N, 1), _f32)))

_layer = pl.pallas_call(
    _layer_body,
    out_shape=jax.ShapeDtypeStruct((N, H), _f32))

_layer4 = pl.pallas_call(
    _layer4_body,
    out_shape=jax.ShapeDtypeStruct((N, H), _f32))

_final = pl.pallas_call(
    _final_body,
    out_shape=jax.ShapeDtypeStruct((100, 100), _f32))


def kernel(x, edge_index, W0, b0, W1, b1, W2, b2, W3, b3, W4, b4):
    src, dst = edge_index[0], edge_index[1]
    pad = EP - E
    srcp = jnp.concatenate(
        [src, jnp.zeros((pad,), jnp.int32)]).reshape(NCH, CH)
    pad_dst = N + (jnp.arange(pad, dtype=jnp.int32) % 128)
    dstp = jnp.concatenate([dst, pad_dst]).reshape(NCH, CH)
    zeros1 = jnp.zeros((NR,), _f32)
    zeros2 = jnp.zeros((NR, H), _f32)

    degp = _sc_degree(dstp, zeros1)
    g, dinv = _dense0(x, W0, degp.reshape(NC, NR, 1))

    for b, W in ((b0, W1), (b1, W2), (b2, W3)):
        acc = _sc_aggregate(g, srcp, dstp, zeros2)
        g = _layer(acc, g, dinv, b.reshape(1, H), W)

    acc = _sc_aggregate(g, srcp, dstp, zeros2)
    g = _layer4(acc, g, dinv, b3.reshape(1, H))

    acc = _sc_aggregate(g, srcp, dstp, zeros2)
    out = _final(acc, g, dinv, W4, b4.reshape(1, 1))
    return out.reshape(1, 100, 100)


# speedup vs baseline: 1.2953x; 1.0338x over previous
import functools

import jax
import jax.numpy as jnp
from jax import lax
from jax.experimental import pallas as pl
from jax.experimental.pallas import tpu as pltpu
from jax.experimental.pallas import tpu_sc as plsc

N = 10000
E = 320000
D = 128
H = 16
NC = 2
NS = 16
NW = NC * NS

CH = 128
KB = 8
KC0 = 136
KC1 = 24
KCM = max(KC0, KC1)
NCH = NS * (KC0 + KC1)
EP = NCH * CH
NR = 10240
RPT = NR // NS

_mesh = plsc.VectorSubcoreMesh(core_axis_name="c", subcore_axis_name="s")


@functools.partial(
    pl.kernel,
    out_type=jax.ShapeDtypeStruct((NC, NR), jnp.float32),
    mesh=_mesh,
    scratch_types=[
        pltpu.VMEM((KCM, CH), jnp.int32),
        pltpu.VMEM((CH,), jnp.float32),
        pltpu.VMEM_SHARED((NR,), jnp.float32),
        pltpu.SemaphoreType.DMA,
    ],
)
def _sc_degree(dstp_hbm, zeros1_hbm, out_hbm, dst_v, ones_v, deg_sh, sem):
    c = lax.axis_index("c")
    s = lax.axis_index("s")
    pltpu.sync_copy(zeros1_hbm.at[pl.ds(s * RPT, RPT)],
                    deg_sh.at[pl.ds(s * RPT, RPT)])
    for i in range(CH // 16):
        ones_v[pl.ds(i * 16, 16)] = jnp.ones((16,), jnp.float32)

    def run(kc, cbase):
        if kc == 0:
            return
        pltpu.sync_copy(dstp_hbm.at[pl.ds(cbase, kc)],
                        dst_v.at[pl.ds(0, kc)])
        descs = [pltpu.async_copy(
            ones_v, deg_sh.at[dst_v.at[j]], sem, add=True)
            for j in range(kc)]
        for d in descs:
            d.wait()

    plsc.subcore_barrier()
    lax.cond(c == 0,
             lambda: run(KC0, s * KC0),
             lambda: run(KC1, NS * KC0 + s * KC1))
    plsc.subcore_barrier()
    pltpu.sync_copy(deg_sh.at[pl.ds(s * RPT, RPT)],
                    out_hbm.at[c].at[pl.ds(s * RPT, RPT)])


@functools.partial(
    pl.kernel,
    out_type=jax.ShapeDtypeStruct((NC, NR, H), jnp.float32),
    mesh=_mesh,
    compiler_params=pltpu.CompilerParams(use_tc_tiling_on_sc=False),
    scratch_types=[
        pltpu.VMEM((KCM, CH), jnp.int32),
        pltpu.VMEM((KCM, CH), jnp.int32),
        pltpu.VMEM((2, KB, CH, H), jnp.float32),
        pltpu.VMEM_SHARED((NR, H), jnp.float32),
        pltpu.SemaphoreType.DMA,
        pltpu.SemaphoreType.DMA,
    ],
)
def _sc_aggregate(g_hbm, srcp_hbm, dstp_hbm, zeros2_hbm, out_hbm,
                  src_v, dst_v, rows_v, acc_sh, gsem, ssem):
    c = lax.axis_index("c")
    s = lax.axis_index("s")
    pltpu.sync_copy(zeros2_hbm.at[pl.ds(s * RPT, RPT)],
                    acc_sh.at[pl.ds(s * RPT, RPT)])

    def gather_group(gi, bank, kb):
        return [pltpu.async_copy(
            g_hbm.at[src_v.at[gi * KB + b]], rows_v.at[bank].at[b], gsem)
            for b in range(kb)]

    def scatter_group(gi, bank, kb):
        return [pltpu.async_copy(
            rows_v.at[bank].at[b], acc_sh.at[dst_v.at[gi * KB + b]],
            ssem, add=True)
            for b in range(kb)]

    def run(kc, cbase):
        if kc == 0:
            return
        pltpu.sync_copy(srcp_hbm.at[pl.ds(cbase, kc)],
                        src_v.at[pl.ds(0, kc)])
        pltpu.sync_copy(dstp_hbm.at[pl.ds(cbase, kc)],
                        dst_v.at[pl.ds(0, kc)])
        ng = kc // KB
        gd = gather_group(0, 0, KB)
        pending = None
        for gi in range(ng):
            bank = gi % 2
            for d in gd:
                d.wait()
            if gi + 1 < ng:
                if pending is not None:
                    for d in pending:
                        d.wait()
                gd = gather_group(gi + 1, 1 - bank, KB)
                pending = scatter_group(gi, bank, KB)
            else:
                last = scatter_group(gi, bank, KB)
                for d in pending:
                    d.wait()
                for d in last:
                    d.wait()

    plsc.subcore_barrier()
    lax.cond(c == 0,
             lambda: run(KC0, s * KC0),
             lambda: run(KC1, NS * KC0 + s * KC1))
    plsc.subcore_barrier()
    pltpu.sync_copy(acc_sh.at[pl.ds(s * RPT, RPT)],
                    out_hbm.at[c].at[pl.ds(s * RPT, RPT)])


def _dense0_body(x_ref, w_ref, deg_ref, g_ref, dinv_ref):
    d = deg_ref[0] + deg_ref[1] + 1.0
    dinv = lax.rsqrt(d)[:N]
    h = jnp.dot(x_ref[...], w_ref[...], preferred_element_type=jnp.float32)
    g_ref[...] = dinv * h
    dinv_ref[...] = dinv


def _layer_body(acc_ref, g_ref, dinv_ref, b_ref, w_ref, out_ref):
    dinv = dinv_ref[...]
    f = jnp.maximum(
        dinv * (acc_ref[0, :N] + acc_ref[1, :N] + g_ref[...]) + b_ref[...],
        0.0)
    out_ref[...] = dinv * jnp.dot(f, w_ref[...],
                                  preferred_element_type=jnp.float32)


def _layer4_body(acc_ref, g_ref, dinv_ref, b_ref, out_ref):
    dinv = dinv_ref[...]
    out_ref[...] = dinv * jnp.maximum(
        dinv * (acc_ref[0, :N] + acc_ref[1, :N] + g_ref[...]) + b_ref[...],
        0.0)


def _final_body(acc_ref, g_ref, dinv_ref, w_ref, b_ref, out_ref):
    af = dinv_ref[...] * (acc_ref[0, :N] + acc_ref[1, :N] + g_ref[...])
    z = jnp.dot(af, w_ref[...],
                preferred_element_type=jnp.float32) + b_ref[...]
    z = z.reshape(100, 100)
    m = jnp.max(z, axis=1, keepdims=True)
    e = jnp.exp(z - m)
    out_ref[...] = e / jnp.sum(e, axis=1, keepdims=True)


_f32 = jnp.float32

_dense0 = pl.pallas_call(
    _dense0_body,
    out_shape=(jax.ShapeDtypeStruct((N, H), _f32),
               jax.ShapeDtypeStruct((N, 1), _f32)))

_layer = pl.pallas_call(
    _layer_body,
    out_shape=jax.ShapeDtypeStruct((N, H), _f32))

_layer4 = pl.pallas_call(
    _layer4_body,
    out_shape=jax.ShapeDtypeStruct((N, H), _f32))

_final = pl.pallas_call(
    _final_body,
    out_shape=jax.ShapeDtypeStruct((100, 100), _f32))


def kernel(x, edge_index, W0, b0, W1, b1, W2, b2, W3, b3, W4, b4):
    src, dst = edge_index[0], edge_index[1]
    pad = EP - E
    srcp = jnp.concatenate(
        [src, jnp.zeros((pad,), jnp.int32)]).reshape(NCH, CH)
    pad_dst = N + (jnp.arange(pad, dtype=jnp.int32) % 128)
    dstp = jnp.concatenate([dst, pad_dst]).reshape(NCH, CH)
    zeros1 = jnp.zeros((NR,), _f32)
    zeros2 = jnp.zeros((NR, H), _f32)

    degp = _sc_degree(dstp, zeros1)
    g, dinv = _dense0(x, W0, degp.reshape(NC, NR, 1))

    for b, W in ((b0, W1), (b1, W2), (b2, W3)):
        acc = _sc_aggregate(g, srcp, dstp, zeros2)
        g = _layer(acc, g, dinv, b.reshape(1, H), W)

    acc = _sc_aggregate(g, srcp, dstp, zeros2)
    g = _layer4(acc, g, dinv, b3.reshape(1, H))

    acc = _sc_aggregate(g, srcp, dstp, zeros2)
    out = _final(acc, g, dinv, W4, b4.reshape(1, 1))
    return out.reshape(1, 100, 100)


# speedup vs baseline: 1.3648x; 1.0537x over previous
import functools

import jax
import jax.numpy as jnp
from jax import lax
from jax.experimental import pallas as pl
from jax.experimental.pallas import tpu as pltpu
from jax.experimental.pallas import tpu_sc as plsc

N = 10000
E = 320000
D = 128
H = 16
NC = 2
NS = 16
NW = NC * NS

CH = 128
KB = 8
KC0 = 144
KC1 = 16
KCM = max(KC0, KC1)
NCH = NS * (KC0 + KC1)
EP = NCH * CH
NR = 10240
RPT = NR // NS

_mesh = plsc.VectorSubcoreMesh(core_axis_name="c", subcore_axis_name="s")


@functools.partial(
    pl.kernel,
    out_type=jax.ShapeDtypeStruct((NC, NR), jnp.float32),
    mesh=_mesh,
    scratch_types=[
        pltpu.VMEM((KCM, CH), jnp.int32),
        pltpu.VMEM((CH,), jnp.float32),
        pltpu.VMEM_SHARED((NR,), jnp.float32),
        pltpu.SemaphoreType.DMA,
    ],
)
def _sc_degree(dstp_hbm, zeros1_hbm, out_hbm, dst_v, ones_v, deg_sh, sem):
    c = lax.axis_index("c")
    s = lax.axis_index("s")
    pltpu.sync_copy(zeros1_hbm.at[pl.ds(s * RPT, RPT)],
                    deg_sh.at[pl.ds(s * RPT, RPT)])
    for i in range(CH // 16):
        ones_v[pl.ds(i * 16, 16)] = jnp.ones((16,), jnp.float32)

    def run(kc, cbase):
        if kc == 0:
            return
        pltpu.sync_copy(dstp_hbm.at[pl.ds(cbase, kc)],
                        dst_v.at[pl.ds(0, kc)])
        descs = [pltpu.async_copy(
            ones_v, deg_sh.at[dst_v.at[j]], sem, add=True)
            for j in range(kc)]
        for d in descs:
            d.wait()

    plsc.subcore_barrier()
    lax.cond(c == 0,
             lambda: run(KC0, s * KC0),
             lambda: run(KC1, NS * KC0 + s * KC1))
    plsc.subcore_barrier()
    pltpu.sync_copy(deg_sh.at[pl.ds(s * RPT, RPT)],
                    out_hbm.at[c].at[pl.ds(s * RPT, RPT)])


@functools.partial(
    pl.kernel,
    out_type=jax.ShapeDtypeStruct((NC, NR, H), jnp.float32),
    mesh=_mesh,
    compiler_params=pltpu.CompilerParams(use_tc_tiling_on_sc=False),
    scratch_types=[
        pltpu.VMEM((KCM, CH), jnp.int32),
        pltpu.VMEM((KCM, CH), jnp.int32),
        pltpu.VMEM((2, KB, CH, H), jnp.float32),
        pltpu.VMEM_SHARED((NR, H), jnp.float32),
        pltpu.SemaphoreType.DMA,
        pltpu.SemaphoreType.DMA,
    ],
)
def _sc_aggregate(g_hbm, srcp_hbm, dstp_hbm, zeros2_hbm, out_hbm,
                  src_v, dst_v, rows_v, acc_sh, gsem, ssem):
    c = lax.axis_index("c")
    s = lax.axis_index("s")
    pltpu.sync_copy(zeros2_hbm.at[pl.ds(s * RPT, RPT)],
                    acc_sh.at[pl.ds(s * RPT, RPT)])

    def gather_group(gi, bank, kb):
        return [pltpu.async_copy(
            g_hbm.at[src_v.at[gi * KB + b]], rows_v.at[bank].at[b], gsem)
            for b in range(kb)]

    def scatter_group(gi, bank, kb):
        return [pltpu.async_copy(
            rows_v.at[bank].at[b], acc_sh.at[dst_v.at[gi * KB + b]],
            ssem, add=True)
            for b in range(kb)]

    def run(kc, cbase):
        if kc == 0:
            return
        pltpu.sync_copy(srcp_hbm.at[pl.ds(cbase, kc)],
                        src_v.at[pl.ds(0, kc)])
        pltpu.sync_copy(dstp_hbm.at[pl.ds(cbase, kc)],
                        dst_v.at[pl.ds(0, kc)])
        ng = kc // KB
        gd = gather_group(0, 0, KB)
        pending = None
        for gi in range(ng):
            bank = gi % 2
            for d in gd:
                d.wait()
            if gi + 1 < ng:
                if pending is not None:
                    for d in pending:
                        d.wait()
                gd = gather_group(gi + 1, 1 - bank, KB)
                pending = scatter_group(gi, bank, KB)
            else:
                last = scatter_group(gi, bank, KB)
                for d in pending:
                    d.wait()
                for d in last:
                    d.wait()

    plsc.subcore_barrier()
    lax.cond(c == 0,
             lambda: run(KC0, s * KC0),
             lambda: run(KC1, NS * KC0 + s * KC1))
    plsc.subcore_barrier()
    pltpu.sync_copy(acc_sh.at[pl.ds(s * RPT, RPT)],
                    out_hbm.at[c].at[pl.ds(s * RPT, RPT)])


def _dense0_body(x_ref, w_ref, deg_ref, g_ref, dinv_ref):
    d = deg_ref[0] + deg_ref[1] + 1.0
    dinv = lax.rsqrt(d)[:N]
    h = jnp.dot(x_ref[...], w_ref[...], preferred_element_type=jnp.float32)
    g_ref[...] = dinv * h
    dinv_ref[...] = dinv


def _layer_body(acc_ref, g_ref, dinv_ref, b_ref, w_ref, out_ref):
    dinv = dinv_ref[...]
    f = jnp.maximum(
        dinv * (acc_ref[0, :N] + acc_ref[1, :N] + g_ref[...]) + b_ref[...],
        0.0)
    out_ref[...] = dinv * jnp.dot(f, w_ref[...],
                                  preferred_element_type=jnp.float32)


def _layer4_body(acc_ref, g_ref, dinv_ref, b_ref, out_ref):
    dinv = dinv_ref[...]
    out_ref[...] = dinv * jnp.maximum(
        dinv * (acc_ref[0, :N] + acc_ref[1, :N] + g_ref[...]) + b_ref[...],
        0.0)


def _final_body(acc_ref, g_ref, dinv_ref, w_ref, b_ref, out_ref):
    af = dinv_ref[...] * (acc_ref[0, :N] + acc_ref[1, :N] + g_ref[...])
    z = jnp.dot(af, w_ref[...],
                preferred_element_type=jnp.float32) + b_ref[...]
    z = z.reshape(100, 100)
    m = jnp.max(z, axis=1, keepdims=True)
    e = jnp.exp(z - m)
    out_ref[...] = e / jnp.sum(e, axis=1, keepdims=True)


_f32 = jnp.float32

_dense0 = pl.pallas_call(
    _dense0_body,
    out_shape=(jax.ShapeDtypeStruct((N, H), _f32),
               jax.ShapeDtypeStruct((N, 1), _f32)))

_layer = pl.pallas_call(
    _layer_body,
    out_shape=jax.ShapeDtypeStruct((N, H), _f32))

_layer4 = pl.pallas_call(
    _layer4_body,
    out_shape=jax.ShapeDtypeStruct((N, H), _f32))

_final = pl.pallas_call(
    _final_body,
    out_shape=jax.ShapeDtypeStruct((100, 100), _f32))


def kernel(x, edge_index, W0, b0, W1, b1, W2, b2, W3, b3, W4, b4):
    src, dst = edge_index[0], edge_index[1]
    pad = EP - E
    srcp = jnp.concatenate(
        [src, jnp.zeros((pad,), jnp.int32)]).reshape(NCH, CH)
    pad_dst = N + (jnp.arange(pad, dtype=jnp.int32) % 128)
    dstp = jnp.concatenate([dst, pad_dst]).reshape(NCH, CH)
    zeros1 = jnp.zeros((NR,), _f32)
    zeros2 = jnp.zeros((NR, H), _f32)

    degp = _sc_degree(dstp, zeros1)
    g, dinv = _dense0(x, W0, degp.reshape(NC, NR, 1))

    for b, W in ((b0, W1), (b1, W2), (b2, W3)):
        acc = _sc_aggregate(g, srcp, dstp, zeros2)
        g = _layer(acc, g, dinv, b.reshape(1, H), W)

    acc = _sc_aggregate(g, srcp, dstp, zeros2)
    g = _layer4(acc, g, dinv, b3.reshape(1, H))

    acc = _sc_aggregate(g, srcp, dstp, zeros2)
    out = _final(acc, g, dinv, W4, b4.reshape(1, 1))
    return out.reshape(1, 100, 100)
